# Initial kernel scaffold; baseline (speedup 1.0000x reference)
import jax
import jax.numpy as jnp
from jax.experimental import pallas as pl


def kernel(x, edge_index, node_type, edge_type, edge_attr, W_msg1, W_self1, W_edge1, nt_emb1, et_emb1, W_msg2, W_self2, W_edge2, nt_emb2, et_emb2, w_v, w_e, w_ea):
    raise NotImplementedError("write your pallas kernel here")


# speedup vs baseline: 2.8613x; 2.8613x over previous
import functools

import jax
import jax.numpy as jnp
from jax import lax
from jax.experimental import pallas as pl
from jax.experimental.pallas import tpu as pltpu
from jax.experimental.pallas import tpu_sc as plsc

N = 10000
E = 320000
D = 128
H = 128
DE = 16
NT = 3
ET = 4

NC = 2
NS = 16
NW = NC * NS
EPW = E // NW
CB = 80
NCH = EPW // CB
RPT = N // NS
KH = H // 16
BE = 4000


def _edge_feat_body(ea_ref, et_ref, wcat_ref, eat1_ref, eat2_ref):
    ea = ea_ref[...]
    oh = (et_ref[...] == lax.broadcasted_iota(jnp.int32, (BE, ET), 1)
          ).astype(jnp.float32)
    a = jnp.concatenate([ea, oh], axis=1)
    r = jnp.dot(a, wcat_ref[...], preferred_element_type=jnp.float32)
    eat1_ref[...] = r[:, :H]
    eat2_ref[...] = r[:, H:]


@jax.jit
def _edge_feat(edge_attr, et2d, wcat):
    return pl.pallas_call(
        _edge_feat_body,
        grid=(E // BE,),
        in_specs=[
            pl.BlockSpec((BE, DE), lambda i: (i, 0)),
            pl.BlockSpec((BE, 1), lambda i: (i, 0)),
            pl.BlockSpec((DE + ET, 2 * H), lambda i: (0, 0)),
        ],
        out_specs=[
            pl.BlockSpec((BE, H), lambda i: (i, 0)),
            pl.BlockSpec((BE, H), lambda i: (i, 0)),
        ],
        out_shape=[
            jax.ShapeDtypeStruct((E, H), jnp.float32),
            jax.ShapeDtypeStruct((E, H), jnp.float32),
        ],
    )(edge_attr, et2d, wcat)


def _node_pre_body(x_ref, nt_ref, wm_ref, wsn_ref, xm_ref, xsn_ref):
    xv = x_ref[...]
    xm_ref[...] = jnp.dot(xv, wm_ref[...], preferred_element_type=jnp.float32)
    oh = (nt_ref[...] == lax.broadcasted_iota(jnp.int32, (N, NT), 1)
          ).astype(jnp.float32)
    a = jnp.concatenate([xv, oh], axis=1)
    xsn_ref[...] = jnp.dot(a, wsn_ref[...], preferred_element_type=jnp.float32)


@jax.jit
def _node_pre(x, nt2d, wm, wsn):
    return pl.pallas_call(
        _node_pre_body,
        out_shape=[
            jax.ShapeDtypeStruct((N, H), jnp.float32),
            jax.ShapeDtypeStruct((N, H), jnp.float32),
        ],
    )(x, nt2d, wm, wsn)


def _node_mid_body(xsn_ref, a0_ref, a1_ref, nt_ref, wm_ref, wsn_ref,
                   xm2_ref, xsn2_ref):
    h1 = jnp.maximum(xsn_ref[...] + a0_ref[...] + a1_ref[...], 0.0)
    xm2_ref[...] = jnp.dot(h1, wm_ref[...], preferred_element_type=jnp.float32)
    oh = (nt_ref[...] == lax.broadcasted_iota(jnp.int32, (N, NT), 1)
          ).astype(jnp.float32)
    a = jnp.concatenate([h1, oh], axis=1)
    xsn2_ref[...] = jnp.dot(a, wsn_ref[...], preferred_element_type=jnp.float32)


@jax.jit
def _node_mid(xsn1, a0, a1, nt2d, wm2, wsn2):
    return pl.pallas_call(
        _node_mid_body,
        out_shape=[
            jax.ShapeDtypeStruct((N, H), jnp.float32),
            jax.ShapeDtypeStruct((N, H), jnp.float32),
        ],
    )(xsn1, a0, a1, nt2d, wm2, wsn2)


def _node_fin_body(xsn_ref, a0_ref, a1_ref, wv_ref, wet_ref,
                   v_ref, h2_ref, h2w_ref):
    h2 = jnp.maximum(xsn_ref[...] + a0_ref[...] + a1_ref[...], 0.0)
    v_ref[...] = jnp.dot(h2, wv_ref[...], preferred_element_type=jnp.float32)
    h2_ref[...] = h2
    h2w_ref[...] = h2 * wet_ref[...]


@jax.jit
def _node_fin(xsn2, a0, a1, wv, wet):
    return pl.pallas_call(
        _node_fin_body,
        out_shape=[
            jax.ShapeDtypeStruct((N, 1), jnp.float32),
            jax.ShapeDtypeStruct((N, H), jnp.float32),
            jax.ShapeDtypeStruct((N, H), jnp.float32),
        ],
    )(xsn2, a0, a1, wv, wet)


NOCT = N // 8
ZIT = -(-NOCT // NS)


def _sc_layer_body(xm_hbm, eat_hbm, src_hbm, dst_hbm, out_hbm,
                   srci, dsti, rows, eatv, aggs, sem_g, sem_e):
    c = lax.axis_index("c")
    s = lax.axis_index("s")
    wid = c * NS + s
    ebase = wid * EPW

    def zrow(i, carry):
        for k in range(KH):
            rows[i, pl.ds(k * 16, 16)] = jnp.zeros((16,), jnp.float32)
        return carry
    lax.fori_loop(0, 8, zrow, 0)

    def zoct(t, carry):
        q = s + NS * t

        @pl.when(q < NOCT)
        def _():
            pltpu.sync_copy(rows.at[pl.ds(0, 8), :],
                            aggs.at[pl.ds(q * 8, 8), :])
        return carry
    lax.fori_loop(0, ZIT, zoct, 0)
    plsc.subcore_barrier()

    def chunk(j, carry):
        base = ebase + j * CB
        pltpu.sync_copy(src_hbm.at[pl.ds(base, CB)], srci)
        pltpu.sync_copy(dst_hbm.at[pl.ds(base, CB)], dsti)
        cg = pltpu.async_copy(xm_hbm.at[srci], rows, sem_g)
        ce = pltpu.async_copy(eat_hbm.at[pl.ds(base, CB), :], eatv, sem_e)
        cg.wait()
        ce.wait()

        def cmp(i, cc):
            for k in range(KH):
                sl = pl.ds(k * 16, 16)
                rows[i, sl] = jnp.maximum(rows[i, sl] + eatv[i, sl], 0.0)
            return cc
        lax.fori_loop(0, CB, cmp, 0)
        pltpu.sync_copy(rows, aggs.at[dsti], add=True)
        return carry
    lax.fori_loop(0, NCH, chunk, 0)

    plsc.subcore_barrier()

    def coct(t, carry):
        q = s + NS * t

        @pl.when(q < NOCT)
        def _():
            pltpu.sync_copy(aggs.at[pl.ds(q * 8, 8), :],
                            out_hbm.at[pl.ds(c * N + q * 8, 8), :])
        return carry
    lax.fori_loop(0, ZIT, coct, 0)


@functools.lru_cache(maxsize=None)
def _sc_layer_fn():
    mesh = plsc.VectorSubcoreMesh(core_axis_name="c", subcore_axis_name="s")
    return pl.kernel(
        _sc_layer_body,
        out_type=jax.ShapeDtypeStruct((NC * N, H), jnp.float32),
        mesh=mesh,
        scratch_types=[
            pltpu.VMEM((CB,), jnp.int32),
            pltpu.VMEM((CB,), jnp.int32),
            pltpu.VMEM((CB, H), jnp.float32),
            pltpu.VMEM((CB, H), jnp.float32),
            pltpu.VMEM_SHARED((N, H), jnp.float32),
            pltpu.SemaphoreType.DMA,
            pltpu.SemaphoreType.DMA,
        ],
    )


def _lane_rot(x, lane, sh):
    dn = lax.GatherDimensionNumbers(
        offset_dims=(), collapsed_slice_dims=(0,), start_index_map=(0,))
    return lax.gather(x, ((lane + sh) & 15)[:, None], dn, (1,),
                      mode=lax.GatherScatterMode.PROMISE_IN_BOUNDS)


def _sc_ehead_body(h2_hbm, h2w_hbm, ea_hbm, wea_hbm, src_hbm, dst_hbm,
                   out_hbm, srci, dsti, av, bv, eav, outv, wear,
                   sem_a, sem_b, sem_c):
    c = lax.axis_index("c")
    s = lax.axis_index("s")
    wid = c * NS + s
    ebase = wid * EPW

    pltpu.sync_copy(wea_hbm, wear)
    wv = wear[...]

    def chunk(j, carry):
        base = ebase + j * CB
        pltpu.sync_copy(src_hbm.at[pl.ds(base, CB)], srci)
        pltpu.sync_copy(dst_hbm.at[pl.ds(base, CB)], dsti)
        ca = pltpu.async_copy(h2_hbm.at[srci], av, sem_a)
        cb = pltpu.async_copy(h2w_hbm.at[dsti], bv, sem_b)
        cc = pltpu.async_copy(ea_hbm.at[pl.ds(base, CB), :], eav, sem_c)
        ca.wait()
        cb.wait()
        cc.wait()
        lane = lax.iota(jnp.int32, 16)

        def group(g, cy):
            vec = jnp.zeros((16,), jnp.float32)
            for i in range(16):
                eidx = g * 16 + i
                acc = eav[eidx, :] * wv
                for k in range(KH):
                    sl = pl.ds(k * 16, 16)
                    acc = acc + av[eidx, sl] * bv[eidx, sl]
                for sh in (8, 4, 2, 1):
                    acc = acc + _lane_rot(acc, lane, sh)
                vec = jnp.where(lane == i, acc, vec)
            outv[pl.ds(g * 16, 16)] = vec
            return cy
        lax.fori_loop(0, CB // 16, group, 0)
        pltpu.sync_copy(outv, out_hbm.at[pl.ds(base, CB)])
        return carry
    lax.fori_loop(0, NCH, chunk, 0)


@functools.lru_cache(maxsize=None)
def _sc_ehead_fn():
    mesh = plsc.VectorSubcoreMesh(core_axis_name="c", subcore_axis_name="s")
    return pl.kernel(
        _sc_ehead_body,
        out_type=jax.ShapeDtypeStruct((E,), jnp.float32),
        mesh=mesh,
        scratch_types=[
            pltpu.VMEM((CB,), jnp.int32),
            pltpu.VMEM((CB,), jnp.int32),
            pltpu.VMEM((CB, H), jnp.float32),
            pltpu.VMEM((CB, H), jnp.float32),
            pltpu.VMEM((CB, DE), jnp.float32),
            pltpu.VMEM((CB,), jnp.float32),
            pltpu.VMEM((DE,), jnp.float32),
            pltpu.SemaphoreType.DMA,
            pltpu.SemaphoreType.DMA,
            pltpu.SemaphoreType.DMA,
        ],
    )


def kernel(x, edge_index, node_type, edge_type, edge_attr,
           W_msg1, W_self1, W_edge1, nt_emb1, et_emb1,
           W_msg2, W_self2, W_edge2, nt_emb2, et_emb2,
           w_v, w_e, w_ea):
    src1d = edge_index[0]
    dst1d = edge_index[1]
    et2d = edge_type.reshape(E, 1)
    nt2d = node_type.reshape(N, 1)

    wcat = jnp.concatenate(
        [jnp.concatenate([W_edge1, W_edge2], axis=1),
         jnp.concatenate([et_emb1, et_emb2], axis=1)], axis=0)
    wsn1 = jnp.concatenate([W_self1, nt_emb1], axis=0)
    wsn2 = jnp.concatenate([W_self2, nt_emb2], axis=0)

    eat1, eat2 = _edge_feat(edge_attr, et2d, wcat)
    xm1, xsn1 = _node_pre(x, nt2d, W_msg1, wsn1)
    agg1 = _sc_layer_fn()(xm1, eat1, src1d, dst1d)
    xm2, xsn2 = _node_mid(xsn1, agg1[:N], agg1[N:], nt2d, W_msg2, wsn2)
    agg2 = _sc_layer_fn()(xm2, eat2, src1d, dst1d)
    v, h2, h2w = _node_fin(xsn2, agg2[:N], agg2[N:], w_v, w_e.reshape(1, H))
    e_flat = _sc_ehead_fn()(h2, h2w, edge_attr, w_ea.reshape(DE),
                            src1d, dst1d)
    return (v, e_flat.reshape(E, 1))
